# baseline (speedup 1.0000x reference)
import jax
import jax.numpy as jnp
from jax.experimental import pallas as pl
from jax.experimental.pallas import tpu as pltpu
from jax.experimental.pallas import tpu_sc as plsc

F32 = jnp.float32

_N_TOK = 50176
_VQ_TILE = 224
_VQ_GRID = _N_TOK // _VQ_TILE
_K = 1024
_D = 32
_SC_WIN = 128


def _conv(x, w, b, stride, pad):
    out = jax.lax.conv_general_dilated(x, w, (stride, stride), [(pad, pad), (pad, pad)], dimension_numbers=('NCHW', 'OIHW', 'NCHW'))
    return out + b[None, :, None, None]


def _conv_t(x, w, b, stride, pad):
    k = w.shape[2]
    p = k - 1 - pad
    out = jax.lax.conv_general_dilated(x, w, (1, 1), [(p, p), (p, p)], lhs_dilation=(stride, stride), dimension_numbers=('NCHW', 'OIHW', 'NCHW'))
    return out + b[None, :, None, None]


def _vq_body(flat_ref, cbt_ref, idx_ref, loss_ref):
    flat = flat_ref[...]
    cbt = cbt_ref[...]
    g = jnp.dot(flat, cbt, preferred_element_type=F32)
    s_c = jnp.sum(cbt * cbt, axis=0)
    s_z = jnp.sum(flat * flat, axis=1)
    dist = s_z[:, None] + s_c[None, :] - 2.0 * g
    m = jnp.min(dist, axis=1)
    iota = jax.lax.broadcasted_iota(jnp.int32, dist.shape, 1)
    idx = jnp.min(jnp.where(dist == m[:, None], iota, jnp.int32(2 ** 30)), axis=1)
    idx_ref[0, 0, :] = idx

    @pl.when(pl.program_id(0) == 0)
    def _():
        loss_ref[0, 0] = 0.0

    loss_ref[0, 0] += jnp.sum(m)


def _vq_argmin(flat, codebook):
    idx3, losssum = pl.pallas_call(
        _vq_body,
        grid=(_VQ_GRID,),
        in_specs=[
            pl.BlockSpec((_VQ_TILE, _D), lambda i: (i, 0)),
            pl.BlockSpec((_D, _K), lambda i: (0, 0)),
        ],
        out_specs=[
            pl.BlockSpec((1, 1, _VQ_TILE), lambda i: (i, 0, 0)),
            pl.BlockSpec((1, 1), lambda i: (0, 0), memory_space=pltpu.SMEM),
        ],
        out_shape=[
            jax.ShapeDtypeStruct((_VQ_GRID, 1, _VQ_TILE), jnp.int32),
            jax.ShapeDtypeStruct((1, 1), F32),
        ],
    )(flat, codebook.T)
    return idx3.reshape(-1), losssum[0, 0]


def _sc_gather(codebook, idx):
    n0 = idx.shape[0]
    unit = _SC_WIN * 32
    n = ((n0 + unit - 1) // unit) * unit
    idx2 = jnp.pad(idx, (0, n - n0)).reshape(1, n)
    cb_pad = jnp.pad(codebook, ((0, 0), (0, 128 - _D)))
    mesh = plsc.VectorSubcoreMesh(core_axis_name="c", subcore_axis_name="s")

    @pl.kernel(out_type=jax.ShapeDtypeStruct((n, 128), F32), mesh=mesh)
    def kern(cb_hbm, i_hbm, o_hbm):
        def body(i_vmem, o_vmem):
            pltpu.sync_copy(cb_hbm.at[i_vmem.at[0]], o_vmem)

        pltpu.emit_pipeline(
            body,
            grid=(n // _SC_WIN,),
            in_specs=[pl.BlockSpec((1, _SC_WIN), lambda i: (0, i))],
            out_specs=[pl.BlockSpec((_SC_WIN, 128), lambda i: (i, 0))],
            core_axis_name=("c", "s"),
            dimension_semantics=(pltpu.PARALLEL,),
        )(i_hbm, o_hbm)

    return kern(cb_pad, idx2)[:n0, :_D]


def kernel(x, w1, b1, w2, b2, w3, b3, codebook, dw1, db1, dw2, db2, dw3, db3):
    z = jax.nn.relu(_conv(x, w1, b1, 2, 1))
    z = jax.nn.relu(_conv(z, w2, b2, 2, 1))
    z = _conv(z, w3, b3, 1, 1)

    flat = jnp.transpose(z, (0, 2, 3, 1)).reshape(-1, _D)
    idx, losssum = _vq_argmin(flat, codebook)
    vq_loss = 1.25 * losssum / (_N_TOK * _D)
    quantized = _sc_gather(codebook, idx)

    qz = jnp.transpose(quantized.reshape(16, 56, 56, _D), (0, 3, 1, 2))
    y = jax.nn.relu(_conv_t(qz, dw1, db1, 1, 1))
    y = jax.nn.relu(_conv_t(y, dw2, db2, 2, 1))
    y = _conv_t(y, dw3, db3, 2, 1)
    return (y, vq_loss)


# speedup vs baseline: 1.0002x; 1.0002x over previous
import jax
import jax.numpy as jnp
from jax.experimental import pallas as pl
from jax.experimental.pallas import tpu as pltpu
from jax.experimental.pallas import tpu_sc as plsc

F32 = jnp.float32

_N_TOK = 50176
_VQ_TILE = 224
_VQ_GRID = _N_TOK // _VQ_TILE
_K = 1024
_D = 32
_SC_WIN = 128


def _conv(x, w, b, stride, pad):
    out = jax.lax.conv_general_dilated(x, w, (stride, stride), [(pad, pad), (pad, pad)], dimension_numbers=('NCHW', 'OIHW', 'NCHW'))
    return out + b[None, :, None, None]


def _conv_t(x, w, b, stride, pad):
    k = w.shape[2]
    p = k - 1 - pad
    out = jax.lax.conv_general_dilated(x, w, (1, 1), [(p, p), (p, p)], lhs_dilation=(stride, stride), dimension_numbers=('NCHW', 'OIHW', 'NCHW'))
    return out + b[None, :, None, None]


def _vq_body(flat_ref, cbt_ref, idx_ref, loss_ref):
    flat = flat_ref[...]
    cbt = cbt_ref[...]
    g = jnp.dot(flat, cbt, preferred_element_type=F32)
    s_c = jnp.sum(cbt * cbt, axis=0)
    s_z = jnp.sum(flat * flat, axis=1)
    dist = s_z[:, None] + s_c[None, :] - 2.0 * g
    m = jnp.min(dist, axis=1)
    iota = jax.lax.broadcasted_iota(jnp.int32, dist.shape, 1)
    idx = jnp.min(jnp.where(dist == m[:, None], iota, jnp.int32(2 ** 30)), axis=1)
    idx_ref[0, 0, :] = idx

    @pl.when(pl.program_id(0) == 0)
    def _():
        loss_ref[0, 0] = 0.0

    loss_ref[0, 0] += jnp.sum(m)


def _vq_argmin(flat, codebook):
    idx3, losssum = pl.pallas_call(
        _vq_body,
        grid=(_VQ_GRID,),
        in_specs=[
            pl.BlockSpec((_VQ_TILE, _D), lambda i: (i, 0)),
            pl.BlockSpec((_D, _K), lambda i: (0, 0)),
        ],
        out_specs=[
            pl.BlockSpec((1, 1, _VQ_TILE), lambda i: (i, 0, 0)),
            pl.BlockSpec((1, 1), lambda i: (0, 0), memory_space=pltpu.SMEM),
        ],
        out_shape=[
            jax.ShapeDtypeStruct((_VQ_GRID, 1, _VQ_TILE), jnp.int32),
            jax.ShapeDtypeStruct((1, 1), F32),
        ],
    )(flat, codebook.T)
    return idx3.reshape(-1), losssum[0, 0]


def _sc_gather(codebook, idx):
    n0 = idx.shape[0]
    unit = _SC_WIN * 32
    n = ((n0 + unit - 1) // unit) * unit
    idx2 = jnp.pad(idx, (0, n - n0)).reshape(1, n)
    cb_pad = jnp.pad(codebook, ((0, 0), (0, 128 - _D)))
    mesh = plsc.VectorSubcoreMesh(core_axis_name="c", subcore_axis_name="s")

    per_unit = n // 32
    steps = per_unit // _SC_WIN

    @pl.kernel(
        out_type=jax.ShapeDtypeStruct((n, 128), F32),
        mesh=mesh,
        scratch_types=[
            pltpu.VMEM((1, per_unit), jnp.int32),
            pltpu.VMEM((_SC_WIN, 128), F32),
            pltpu.SemaphoreType.DMA,
        ],
    )
    def kern(cb_hbm, i_hbm, o_hbm, iv, gbuf, sem):
        c = jax.lax.axis_index("c")
        s = jax.lax.axis_index("s")
        base = (c * 16 + s) * per_unit
        pltpu.async_copy(i_hbm.at[:, pl.ds(base, per_unit)], iv, sem).wait()

        @pl.loop(0, steps)
        def _(k):
            pltpu.sync_copy(cb_hbm.at[iv.at[0, pl.ds(k * _SC_WIN, _SC_WIN)]], gbuf)
            pltpu.sync_copy(gbuf, o_hbm.at[pl.ds(base + k * _SC_WIN, _SC_WIN)])

    return kern(cb_pad, idx2)[:n0, :_D]


def kernel(x, w1, b1, w2, b2, w3, b3, codebook, dw1, db1, dw2, db2, dw3, db3):
    z = jax.nn.relu(_conv(x, w1, b1, 2, 1))
    z = jax.nn.relu(_conv(z, w2, b2, 2, 1))
    z = _conv(z, w3, b3, 1, 1)

    flat = jnp.transpose(z, (0, 2, 3, 1)).reshape(-1, _D)
    idx, losssum = _vq_argmin(flat, codebook)
    vq_loss = 1.25 * losssum / (_N_TOK * _D)
    quantized = _sc_gather(codebook, idx)

    qz = jnp.transpose(quantized.reshape(16, 56, 56, _D), (0, 3, 1, 2))
    y = jax.nn.relu(_conv_t(qz, dw1, db1, 1, 1))
    y = jax.nn.relu(_conv_t(y, dw2, db2, 2, 1))
    y = _conv_t(y, dw3, db3, 2, 1)
    return (y, vq_loss)


# speedup vs baseline: 1.0106x; 1.0105x over previous
from functools import partial

import jax
import jax.numpy as jnp
from jax.experimental import pallas as pl
from jax.experimental.pallas import tpu as pltpu

F32 = jnp.float32

_B = 16
_N_TOK = 50176
_VQ_TILE = 224
_VQ_GRID = _N_TOK // _VQ_TILE
_K = 1024
_D = 32


def _conv_body(in_ref, w_ref, b_ref, o_ref, *, taps, pout, relu):
    pieces = [in_ref[0, g, off:off + pout, :] for (g, off) in taps]
    t = pieces[0] if len(pieces) == 1 else jnp.concatenate(pieces, axis=-1)
    acc = jnp.dot(t, w_ref[...], preferred_element_type=F32)
    acc = acc + b_ref[...]
    if relu:
        acc = jnp.maximum(acc, 0.0)
    o_ref[0] = acc


def _pconv(xg, wmat, bias, taps, pout, relu):
    b, g, pin, c = xg.shape
    k, cout = wmat.shape
    return pl.pallas_call(
        partial(_conv_body, taps=taps, pout=pout, relu=relu),
        grid=(b,),
        in_specs=[
            pl.BlockSpec((1, g, pin, c), lambda i: (i, 0, 0, 0)),
            pl.BlockSpec((k, cout), lambda i: (0, 0)),
            pl.BlockSpec((1, cout), lambda i: (0, 0)),
        ],
        out_specs=pl.BlockSpec((1, pout, cout), lambda i: (i, 0, 0)),
        out_shape=jax.ShapeDtypeStruct((b, pout, cout), F32),
    )(xg, wmat, bias.reshape(1, cout))


def _unpitch(flat, hout, pitch, wout):
    b, p, c = flat.shape
    full = hout * pitch
    flat = jnp.pad(flat, ((0, 0), (0, full - p), (0, 0)))
    return flat.reshape(b, hout, pitch, c)[:, :, :wout, :]


def _vq_body(flat_ref, cbt_ref, cb_ref, q_ref, loss_ref):
    flat = flat_ref[...]
    cbt = cbt_ref[...]
    g = jnp.dot(flat, cbt, preferred_element_type=F32)
    s_c = jnp.sum(cbt * cbt, axis=0)
    s_z = jnp.sum(flat * flat, axis=1)
    dist = s_z[:, None] + s_c[None, :] - 2.0 * g
    m = jnp.min(dist, axis=1)
    iota = jax.lax.broadcasted_iota(jnp.int32, dist.shape, 1)
    idx = jnp.min(jnp.where(dist == m[:, None], iota, jnp.int32(2 ** 30)), axis=1)
    onehot = (iota == idx[:, None]).astype(F32)
    q_ref[...] = jnp.dot(onehot, cb_ref[...], preferred_element_type=F32)

    @pl.when(pl.program_id(0) == 0)
    def _():
        loss_ref[0, 0] = 0.0

    loss_ref[0, 0] += jnp.sum(m)


def _vq(flat, codebook):
    q, losssum = pl.pallas_call(
        _vq_body,
        grid=(_VQ_GRID,),
        in_specs=[
            pl.BlockSpec((_VQ_TILE, _D), lambda i: (i, 0)),
            pl.BlockSpec((_D, _K), lambda i: (0, 0)),
            pl.BlockSpec((_K, _D), lambda i: (0, 0)),
        ],
        out_specs=[
            pl.BlockSpec((_VQ_TILE, _D), lambda i: (i, 0)),
            pl.BlockSpec((1, 1), lambda i: (0, 0), memory_space=pltpu.SMEM),
        ],
        out_shape=[
            jax.ShapeDtypeStruct((_N_TOK, _D), F32),
            jax.ShapeDtypeStruct((1, 1), F32),
        ],
    )(flat, codebook.T, codebook)
    return q, losssum[0, 0]


def _w_conv1(w1):
    w = jnp.zeros((3, 3, 2, 2, 3, 32), F32)
    for u in range(3):
        for v in range(3):
            for a in range(2):
                for b in range(2):
                    kh = 2 * u + a - 1
                    kw = 2 * v + b - 1
                    if 0 <= kh <= 3 and 0 <= kw <= 3:
                        w = w.at[u, v, a, b, :, :].set(
                            jnp.transpose(w1[:, :, kh, kw], (1, 0)))
    return w.reshape(108, 32)


def _w_convt2(dw, cin, cout):
    w = jnp.zeros((3, 3, cin, 2, 2, cout), F32)
    for u in range(3):
        for v in range(3):
            for a in range(2):
                for b in range(2):
                    if 0 <= u - a <= 1 and 0 <= v - b <= 1:
                        w = w.at[u, v, :, a, b, :].set(
                            jnp.transpose(dw[:, :, 2 * u - a, 2 * v - b], (1, 0)))
    return w.reshape(9 * cin, 4 * cout)


def kernel(x, w1, b1, w2, b2, w3, b3, codebook, dw1, db1, dw2, db2, dw3, db3):
    xs = x.reshape(_B, 3, 112, 2, 112, 2).transpose(0, 2, 4, 3, 5, 1)
    xs = xs.reshape(_B, 112, 112, 12)
    sp = jnp.pad(xs, ((0, 0), (1, 1), (1, 1), (0, 0)))
    xcol = jnp.concatenate(
        [sp[:, u:u + 112, v:v + 112, :] for u in range(3) for v in range(3)],
        axis=-1).reshape(_B, 1, 12544, 108)
    z1 = _pconv(xcol, _w_conv1(w1), b1, [(0, 0)], 12544, True)
    z1 = z1.reshape(_B, 112, 112, 32)

    z1p = jnp.pad(z1, ((0, 0), (1, 1), (1, 1), (0, 0)))
    g4 = z1p.reshape(_B, 57, 2, 57, 2, 32).transpose(0, 2, 4, 1, 3, 5)
    g4 = g4.reshape(_B, 4, 3249, 32)
    taps2 = [((kh % 2) * 2 + (kw % 2), (kh // 2) * 57 + (kw // 2))
             for kh in range(4) for kw in range(4)]
    w2m = jnp.transpose(w2, (2, 3, 1, 0)).reshape(512, 64)
    z2 = _pconv(g4, w2m, b2, taps2, 3191, True)
    z2 = _unpitch(z2, 56, 57, 56)

    z2p = jnp.pad(z2, ((0, 0), (1, 1), (1, 1), (0, 0))).reshape(_B, 1, 3364, 64)
    taps9_58 = [(0, u * 58 + v) for u in range(3) for v in range(3)]
    w3m = jnp.transpose(w3, (2, 3, 1, 0)).reshape(576, 32)
    z3 = _pconv(z2p, w3m, b3, taps9_58, 3246, False)
    z3 = _unpitch(z3, 56, 58, 56)

    flat = z3.reshape(_N_TOK, _D)
    quantized, losssum = _vq(flat, codebook)
    vq_loss = 1.25 * losssum / (_N_TOK * _D)

    qp = jnp.pad(quantized.reshape(_B, 56, 56, _D),
                 ((0, 0), (1, 1), (1, 1), (0, 0))).reshape(_B, 1, 3364, _D)
    dw1m = jnp.transpose(dw1, (2, 3, 1, 0)).reshape(288, 64)
    y1 = _pconv(qp, dw1m, db1, taps9_58, 3246, True)
    y1 = _unpitch(y1, 56, 58, 56)

    y1p = jnp.pad(y1, ((0, 0), (1, 1), (1, 1), (0, 0))).reshape(_B, 1, 3364, 64)
    y2 = _pconv(y1p, _w_convt2(dw2, 64, 32), jnp.tile(db2, 4), taps9_58, 3246, True)
    y2 = _unpitch(y2, 56, 58, 56).reshape(_B, 56, 56, 2, 2, 32)
    y2 = y2.transpose(0, 1, 3, 2, 4, 5).reshape(_B, 112, 112, 32)

    y2p = jnp.pad(y2, ((0, 0), (1, 1), (1, 1), (0, 0))).reshape(_B, 1, 12996, 32)
    taps9_114 = [(0, u * 114 + v) for u in range(3) for v in range(3)]
    y3 = _pconv(y2p, _w_convt2(dw3, 32, 3), jnp.tile(db3, 4), taps9_114, 12766, False)
    y3 = _unpitch(y3, 112, 114, 112).reshape(_B, 112, 112, 2, 2, 3)
    y = y3.transpose(0, 1, 3, 2, 4, 5).reshape(_B, 224, 224, 3).transpose(0, 3, 1, 2)

    return (y, vq_loss)


# speedup vs baseline: 1.3851x; 1.3705x over previous
from functools import partial

import jax
import jax.numpy as jnp
from jax.experimental import pallas as pl
from jax.experimental.pallas import tpu as pltpu

F32 = jnp.float32

_B = 16
_N_TOK = 50176
_VQ_TILE = 448
_VQ_GRID = _N_TOK // _VQ_TILE
_K = 1024
_D = 32


def _conv_body(in_ref, w_ref, b_ref, o_ref, *, taps, pstep, relu):
    base = pl.program_id(1) * pstep
    pieces = [in_ref[0, g, pl.ds(base + off, pstep), :] for (g, off) in taps]
    t = pieces[0] if len(pieces) == 1 else jnp.concatenate(pieces, axis=-1)
    acc = jnp.dot(t, w_ref[...], preferred_element_type=F32)
    acc = acc + b_ref[...]
    if relu:
        acc = jnp.maximum(acc, 0.0)
    o_ref[0] = acc


def _pconv(xg, wmat, bias, taps, pout, relu, split=1):
    b, g, pin, c = xg.shape
    k, cout = wmat.shape
    pstep = pout // split
    return pl.pallas_call(
        partial(_conv_body, taps=taps, pstep=pstep, relu=relu),
        grid=(b, split),
        in_specs=[
            pl.BlockSpec((1, g, pin, c), lambda i, j: (i, 0, 0, 0)),
            pl.BlockSpec((k, cout), lambda i, j: (0, 0)),
            pl.BlockSpec((1, cout), lambda i, j: (0, 0)),
        ],
        out_specs=pl.BlockSpec((1, pstep, cout), lambda i, j: (i, j, 0)),
        out_shape=jax.ShapeDtypeStruct((b, pout, cout), F32),
    )(xg, wmat, bias.reshape(1, cout))


def _unpitch(flat, hout, pitch, wout):
    b, p, c = flat.shape
    full = hout * pitch
    flat = jnp.pad(flat, ((0, 0), (0, full - p), (0, 0)))
    return flat.reshape(b, hout, pitch, c)[:, :, :wout, :]


def _vq_body(flat_ref, cbt_ref, cb_ref, q_ref, loss_ref):
    flat = flat_ref[...]
    cbt = cbt_ref[...]
    g = jnp.dot(flat, cbt, preferred_element_type=F32)
    s_c = jnp.sum(cbt * cbt, axis=0)
    s_z = jnp.sum(flat * flat, axis=1)
    dist = s_z[:, None] + s_c[None, :] - 2.0 * g
    m = jnp.min(dist, axis=1)
    iota = jax.lax.broadcasted_iota(jnp.int32, dist.shape, 1)
    idx = jnp.min(jnp.where(dist == m[:, None], iota, jnp.int32(2 ** 30)), axis=1)
    onehot = (iota == idx[:, None]).astype(F32)
    q_ref[...] = jnp.dot(onehot, cb_ref[...], preferred_element_type=F32)

    @pl.when(pl.program_id(0) == 0)
    def _():
        loss_ref[0, 0] = 0.0

    loss_ref[0, 0] += jnp.sum(m)


def _vq(flat, codebook):
    q, losssum = pl.pallas_call(
        _vq_body,
        grid=(_VQ_GRID,),
        in_specs=[
            pl.BlockSpec((_VQ_TILE, _D), lambda i: (i, 0)),
            pl.BlockSpec((_D, _K), lambda i: (0, 0)),
            pl.BlockSpec((_K, _D), lambda i: (0, 0)),
        ],
        out_specs=[
            pl.BlockSpec((_VQ_TILE, _D), lambda i: (i, 0)),
            pl.BlockSpec((1, 1), lambda i: (0, 0), memory_space=pltpu.SMEM),
        ],
        out_shape=[
            jax.ShapeDtypeStruct((_N_TOK, _D), F32),
            jax.ShapeDtypeStruct((1, 1), F32),
        ],
    )(flat, codebook.T, codebook)
    return q, losssum[0, 0]


def _w_conv1(w1):
    w = jnp.zeros((3, 3, 2, 2, 3, 32), F32)
    for u in range(3):
        for v in range(3):
            for a in range(2):
                for b in range(2):
                    kh = 2 * u + a - 1
                    kw = 2 * v + b - 1
                    if 0 <= kh <= 3 and 0 <= kw <= 3:
                        w = w.at[u, v, a, b, :, :].set(
                            jnp.transpose(w1[:, :, kh, kw], (1, 0)))
    return w.reshape(108, 32)


def _w_convt2(dw, cin, cout):
    w = jnp.zeros((3, 3, cin, 2, 2, cout), F32)
    for u in range(3):
        for v in range(3):
            for a in range(2):
                for b in range(2):
                    if 0 <= u - a <= 1 and 0 <= v - b <= 1:
                        w = w.at[u, v, :, a, b, :].set(
                            jnp.transpose(dw[:, :, 2 * u - a, 2 * v - b], (1, 0)))
    return w.reshape(9 * cin, 4 * cout)


def kernel(x, w1, b1, w2, b2, w3, b3, codebook, dw1, db1, dw2, db2, dw3, db3):
    xs = x.reshape(_B, 3, 112, 2, 112, 2).transpose(0, 2, 4, 3, 5, 1)
    xs = xs.reshape(_B, 112, 112, 12)
    sp = jnp.pad(xs, ((0, 0), (1, 1), (1, 1), (0, 0))).reshape(_B, 1, 12996, 12)
    sp = jnp.pad(sp, ((0, 0), (0, 0), (0, 4), (0, 0)))
    taps9_114 = [(0, u * 114 + v) for u in range(3) for v in range(3)]
    z1 = _pconv(sp, _w_conv1(w1), b1, taps9_114, 12768, True, split=4)
    z1 = _unpitch(z1, 112, 114, 112)

    z1p = jnp.pad(z1, ((0, 0), (1, 1), (1, 1), (0, 0)))
    g4 = z1p.reshape(_B, 57, 2, 57, 2, 32).transpose(0, 2, 4, 1, 3, 5)
    g4 = g4.reshape(_B, 4, 3249, 32)
    taps2 = [((kh % 2) * 2 + (kw % 2), (kh // 2) * 57 + (kw // 2))
             for kh in range(4) for kw in range(4)]
    w2m = jnp.transpose(w2, (2, 3, 1, 0)).reshape(512, 64)
    z2 = _pconv(g4, w2m, b2, taps2, 3191, True)
    z2 = _unpitch(z2, 56, 57, 56)

    z2p = jnp.pad(z2, ((0, 0), (1, 1), (1, 1), (0, 0))).reshape(_B, 1, 3364, 64)
    taps9_58 = [(0, u * 58 + v) for u in range(3) for v in range(3)]
    w3m = jnp.transpose(w3, (2, 3, 1, 0)).reshape(576, 32)
    z3 = _pconv(z2p, w3m, b3, taps9_58, 3246, False)
    z3 = _unpitch(z3, 56, 58, 56)

    flat = z3.reshape(_N_TOK, _D)
    quantized, losssum = _vq(flat, codebook)
    vq_loss = 1.25 * losssum / (_N_TOK * _D)

    qp = jnp.pad(quantized.reshape(_B, 56, 56, _D),
                 ((0, 0), (1, 1), (1, 1), (0, 0))).reshape(_B, 1, 3364, _D)
    dw1m = jnp.transpose(dw1, (2, 3, 1, 0)).reshape(288, 64)
    y1 = _pconv(qp, dw1m, db1, taps9_58, 3246, True)
    y1 = _unpitch(y1, 56, 58, 56)

    y1p = jnp.pad(y1, ((0, 0), (1, 1), (1, 1), (0, 0))).reshape(_B, 1, 3364, 64)
    y2 = _pconv(y1p, _w_convt2(dw2, 64, 32), jnp.tile(db2, 4), taps9_58, 3246, True)
    y2 = _unpitch(y2, 56, 58, 56).reshape(_B, 56, 56, 2, 2, 32)
    y2 = y2.transpose(0, 1, 3, 2, 4, 5).reshape(_B, 112, 112, 32)

    y2p = jnp.pad(y2, ((0, 0), (1, 1), (1, 1), (0, 0))).reshape(_B, 1, 12996, 32)
    y3 = _pconv(y2p, _w_convt2(dw3, 32, 3), jnp.tile(db3, 4), taps9_114, 12766, False)
    y3 = _unpitch(y3, 112, 114, 112).reshape(_B, 112, 112, 2, 2, 3)
    y = y3.transpose(0, 1, 3, 2, 4, 5).reshape(_B, 224, 224, 3).transpose(0, 3, 1, 2)

    return (y, vq_loss)
